# baseline (device time: 73869 ns/iter reference)
import jax
import jax.numpy as jnp
from jax import lax
from jax.experimental import pallas as pl
from jax.experimental.pallas import tpu as pltpu

N_DEV = 16


def kernel(x, W1, W2):
    m, _ = x.shape
    out_n = W2.shape[1]
    rows = m // N_DEV

    def body(x_ref, w1_ref, w2_ref, out_ref, partial_ref, acc_ref, rs_buf,
             rs_send, rs_recv, ag_send, ag_recv):
        my_i = lax.axis_index("i")

        xb = x_ref[...].astype(jnp.bfloat16)
        w1 = w1_ref[...].astype(jnp.bfloat16)
        h = jnp.dot(xb, w1, preferred_element_type=jnp.float32)
        hb = jnp.maximum(h, 0.0).astype(jnp.bfloat16)
        w2 = w2_ref[...].astype(jnp.bfloat16)
        partial_ref[...] = jnp.dot(hb, w2, preferred_element_type=jnp.float32)

        rs = []
        for d in range(1, N_DEV):
            tgt = (my_i + d) % N_DEV
            r = pltpu.make_async_remote_copy(
                src_ref=partial_ref.at[pl.ds(tgt * rows, rows)],
                dst_ref=rs_buf.at[d - 1],
                send_sem=rs_send.at[d - 1],
                recv_sem=rs_recv.at[d - 1],
                device_id=(tgt,),
                device_id_type=pl.DeviceIdType.MESH,
            )
            r.start()
            rs.append(r)

        acc = partial_ref[pl.ds(my_i * rows, rows), :]
        for d in range(1, N_DEV):
            rs[d - 1].wait_recv()
            acc = acc + rs_buf[d - 1]
        acc_ref[...] = acc

        ag = []
        for d in range(1, N_DEV):
            tgt = (my_i + d) % N_DEV
            r = pltpu.make_async_remote_copy(
                src_ref=acc_ref,
                dst_ref=out_ref.at[pl.ds(my_i * rows, rows)],
                send_sem=ag_send.at[d - 1],
                recv_sem=ag_recv.at[d - 1],
                device_id=(tgt,),
                device_id_type=pl.DeviceIdType.MESH,
            )
            r.start()
            ag.append(r)

        out_ref[pl.ds(my_i * rows, rows), :] = acc_ref[...]

        for d in range(1, N_DEV):
            ag[d - 1].wait_recv()
            rs[d - 1].wait_send()
            ag[d - 1].wait_send()

    return pl.pallas_call(
        body,
        out_shape=jax.ShapeDtypeStruct((m, out_n), jnp.float32),
        in_specs=[pl.BlockSpec(memory_space=pltpu.VMEM)] * 3,
        out_specs=pl.BlockSpec(memory_space=pltpu.VMEM),
        scratch_shapes=[
            pltpu.VMEM((m, out_n), jnp.float32),
            pltpu.VMEM((rows, out_n), jnp.float32),
            pltpu.VMEM((N_DEV - 1, rows, out_n), jnp.float32),
            pltpu.SemaphoreType.DMA((N_DEV - 1,)),
            pltpu.SemaphoreType.DMA((N_DEV - 1,)),
            pltpu.SemaphoreType.DMA((N_DEV - 1,)),
            pltpu.SemaphoreType.DMA((N_DEV - 1,)),
        ],
    )(x, W1, W2)


# device time: 45818 ns/iter; 1.6122x vs baseline; 1.6122x over previous
import jax
import jax.numpy as jnp
from jax import lax
from jax.experimental import pallas as pl
from jax.experimental.pallas import tpu as pltpu

N_DEV = 16


def kernel(x, W1, W2):
    m, _ = x.shape
    out_n = W2.shape[1]
    rows = m // N_DEV

    def body(x_ref, w1_ref, w2_ref, out_ref, partial_ref, acc_ref, rs_buf,
             rs_send, rs_recv, ag_send, ag_recv):
        my_i = lax.axis_index("i")

        xb = x_ref[...].astype(jnp.bfloat16)
        w1 = w1_ref[...].astype(jnp.bfloat16)
        h = jnp.dot(xb, w1, preferred_element_type=jnp.float32)
        hb = jnp.maximum(h, 0.0).astype(jnp.bfloat16)
        w2 = w2_ref[...].astype(jnp.bfloat16)
        partial_f32 = jnp.dot(hb, w2, preferred_element_type=jnp.float32)
        partial_ref[...] = partial_f32.astype(jnp.bfloat16)

        rs = []
        for d in range(1, N_DEV):
            tgt = (my_i + d) % N_DEV
            r = pltpu.make_async_remote_copy(
                src_ref=partial_ref.at[pl.ds(tgt * rows, rows)],
                dst_ref=rs_buf.at[d - 1],
                send_sem=rs_send.at[d - 1],
                recv_sem=rs_recv.at[d - 1],
                device_id=(tgt,),
                device_id_type=pl.DeviceIdType.MESH,
            )
            r.start()
            rs.append(r)

        acc = partial_ref[pl.ds(my_i * rows, rows), :].astype(jnp.float32)
        for d in range(1, N_DEV):
            rs[d - 1].wait_recv()
            acc = acc + rs_buf[d - 1].astype(jnp.float32)
        acc_ref[...] = acc.astype(jnp.bfloat16)

        ag = []
        for d in range(1, N_DEV):
            tgt = (my_i + d) % N_DEV
            r = pltpu.make_async_remote_copy(
                src_ref=acc_ref,
                dst_ref=out_ref.at[pl.ds(my_i * rows, rows)],
                send_sem=ag_send.at[d - 1],
                recv_sem=ag_recv.at[d - 1],
                device_id=(tgt,),
                device_id_type=pl.DeviceIdType.MESH,
            )
            r.start()
            ag.append(r)

        out_ref[pl.ds(my_i * rows, rows), :] = acc_ref[...]

        for d in range(1, N_DEV):
            ag[d - 1].wait_recv()
            rs[d - 1].wait_send()
            ag[d - 1].wait_send()

    return pl.pallas_call(
        body,
        out_shape=jax.ShapeDtypeStruct((m, out_n), jnp.bfloat16),
        in_specs=[pl.BlockSpec(memory_space=pltpu.VMEM)] * 3,
        out_specs=pl.BlockSpec(memory_space=pltpu.VMEM),
        scratch_shapes=[
            pltpu.VMEM((m, out_n), jnp.bfloat16),
            pltpu.VMEM((rows, out_n), jnp.bfloat16),
            pltpu.VMEM((N_DEV - 1, rows, out_n), jnp.bfloat16),
            pltpu.SemaphoreType.DMA((N_DEV - 1,)),
            pltpu.SemaphoreType.DMA((N_DEV - 1,)),
            pltpu.SemaphoreType.DMA((N_DEV - 1,)),
            pltpu.SemaphoreType.DMA((N_DEV - 1,)),
        ],
    )(x, W1, W2)
